# baseline (speedup 1.0000x reference)
import functools

import jax
import jax.numpy as jnp
from jax import lax
from jax.experimental import pallas as pl
from jax.experimental.pallas import tpu as pltpu
from jax.experimental.pallas import tpu_sc as plsc

N_NODES = 10000
N_EDGES = 320000
ENC = 64
F32 = jnp.float32

BE = 4000
BN = 2000

NC = 2
NS = 16
NW = NC * NS
CH = 80
PER_W = N_EDGES // NW
NCH = PER_W // CH
UB = 5
CHM = 40
NCHM = PER_W // CHM
UBM = 2
NP = 10240
ZR = 128


def _relu(v):
    return jnp.maximum(v, 0.0)


def _dot(a, b):
    return jnp.dot(a, b, preferred_element_type=F32)


def _node_enc_body(x_ref, nW1_ref, nb1_ref, nW2_ref, nb2_ref, ws_ref, wd_ref,
                   x0_ref, ps_ref, pd_ref):
    xb = x_ref[...]
    h = (xb[:, 0:1] * nW1_ref[0:1, :] + xb[:, 1:2] * nW1_ref[1:2, :]
         + nb1_ref[...])
    x0 = _relu(_dot(_relu(h), nW2_ref[...]) + nb2_ref[...])
    x0_ref[...] = x0
    ps_ref[...] = _dot(x0, ws_ref[...])
    pd_ref[...] = _dot(x0, wd_ref[...])


def _node_encoder(x, nW1, nb1, nW2, nb2, ws0, wd0):
    grid = (N_NODES // BN,)
    wspec = pl.BlockSpec((ENC, ENC), lambda i: (0, 0))
    bspec = pl.BlockSpec((1, ENC), lambda i: (0, 0))
    nspec = pl.BlockSpec((BN, ENC), lambda i: (i, 0))
    return pl.pallas_call(
        _node_enc_body,
        grid=grid,
        in_specs=[pl.BlockSpec((BN, 2), lambda i: (i, 0)),
                  pl.BlockSpec((2, ENC), lambda i: (0, 0)), bspec,
                  wspec, bspec, wspec, wspec],
        out_specs=[nspec, nspec, nspec],
        out_shape=[jax.ShapeDtypeStruct((N_NODES, ENC), F32)] * 3,
    )(x, nW1, nb1, nW2, nb2, ws0, wd0)


def _edge_enc_body(ef_ref, eW1_ref, eb1_ref, eW2_ref, eb2_ref, e0_ref):
    eb = ef_ref[...]
    h = (eb[:, 0:1] * eW1_ref[0:1, :] + eb[:, 1:2] * eW1_ref[1:2, :]
         + eb[:, 2:3] * eW1_ref[2:3, :] + eb1_ref[...])
    e0_ref[...] = _relu(_dot(_relu(h), eW2_ref[...]) + eb2_ref[...])


def _edge_encoder(ef, eW1, eb1, eW2, eb2):
    grid = (N_EDGES // BE,)
    return pl.pallas_call(
        _edge_enc_body,
        grid=grid,
        in_specs=[pl.BlockSpec((BE, 3), lambda i: (i, 0)),
                  pl.BlockSpec((3, ENC), lambda i: (0, 0)),
                  pl.BlockSpec((1, ENC), lambda i: (0, 0)),
                  pl.BlockSpec((ENC, ENC), lambda i: (0, 0)),
                  pl.BlockSpec((1, ENC), lambda i: (0, 0))],
        out_specs=pl.BlockSpec((BE, ENC), lambda i: (i, 0)),
        out_shape=jax.ShapeDtypeStruct((N_EDGES, ENC), F32),
    )(ef, eW1, eb1, eW2, eb2)


def _edge_mlp0_body(e0_ref, gs_ref, gd_ref, w1_ref, b1_ref, w2_ref, b2_ref,
                    out_ref):
    h = _relu(_dot(e0_ref[...], w1_ref[...]) + gs_ref[...] + gd_ref[...]
              + b1_ref[...])
    out_ref[...] = _relu(_dot(h, w2_ref[...]) + b2_ref[...])


def _edge_mlp0(e0, gs, gd, w1, b1, w2, b2):
    grid = (N_EDGES // BE,)
    espec = pl.BlockSpec((BE, ENC), lambda i: (i, 0))
    wspec = pl.BlockSpec((ENC, ENC), lambda i: (0, 0))
    bspec = pl.BlockSpec((1, ENC), lambda i: (0, 0))
    return pl.pallas_call(
        _edge_mlp0_body,
        grid=grid,
        in_specs=[espec, espec, espec, wspec, bspec, wspec, bspec],
        out_specs=espec,
        out_shape=jax.ShapeDtypeStruct((N_EDGES, ENC), F32),
    )(e0, gs, gd, w1, b1, w2, b2)


def _edge_mlp_body(ec_ref, e0_ref, gs_ref, gd_ref, w1c_ref, w1e_ref, b1_ref,
                   w2_ref, b2_ref, out_ref):
    h = _relu(_dot(ec_ref[...], w1c_ref[...]) + _dot(e0_ref[...], w1e_ref[...])
              + gs_ref[...] + gd_ref[...] + b1_ref[...])
    out_ref[...] = _relu(_dot(h, w2_ref[...]) + b2_ref[...])


def _edge_mlp(ec, e0, gs, gd, w1c, w1e, b1, w2, b2):
    grid = (N_EDGES // BE,)
    espec = pl.BlockSpec((BE, ENC), lambda i: (i, 0))
    wspec = pl.BlockSpec((ENC, ENC), lambda i: (0, 0))
    bspec = pl.BlockSpec((1, ENC), lambda i: (0, 0))
    return pl.pallas_call(
        _edge_mlp_body,
        grid=grid,
        in_specs=[espec, espec, espec, espec, wspec, wspec, bspec, wspec,
                  bspec],
        out_specs=espec,
        out_shape=jax.ShapeDtypeStruct((N_EDGES, ENC), F32),
    )(ec, e0, gs, gd, w1c, w1e, b1, w2, b2)


def _node_core(xc_ref, x0_ref, sa0_ref, sa1_ref, sb0_ref, sb1_ref,
               wA_ref, wB_ref, wC_ref, wD_ref, b1_ref, w2_ref, b2_ref):
    sec = sa0_ref[...] + sa1_ref[...]
    se0 = sb0_ref[...] + sb1_ref[...]
    h = _relu(_dot(xc_ref[...], wA_ref[...]) + _dot(x0_ref[...], wB_ref[...])
              + _dot(sec, wC_ref[...]) + _dot(se0, wD_ref[...]) + b1_ref[...])
    return _relu(_dot(h, w2_ref[...]) + b2_ref[...])


def _node_mlp_tab_body(xc_ref, x0_ref, sa0_ref, sa1_ref, sb0_ref, sb1_ref,
                       wA_ref, wB_ref, wC_ref, wD_ref, b1_ref, w2_ref, b2_ref,
                       wsc_ref, ws0_ref, wdc_ref, wd0_ref,
                       xcn_ref, ps_ref, pd_ref):
    xcn = _node_core(xc_ref, x0_ref, sa0_ref, sa1_ref, sb0_ref, sb1_ref,
                     wA_ref, wB_ref, wC_ref, wD_ref, b1_ref, w2_ref, b2_ref)
    xcn_ref[...] = xcn
    x0 = x0_ref[...]
    ps_ref[...] = _dot(xcn, wsc_ref[...]) + _dot(x0, ws0_ref[...])
    pd_ref[...] = _dot(xcn, wdc_ref[...]) + _dot(x0, wd0_ref[...])


def _node_mlp_tab(xc, x0, sa0, sa1, sb0, sb1, wA, wB, wC, wD, b1, w2, b2,
                  wsc, ws0, wdc, wd0):
    grid = (N_NODES // BN,)
    nspec = pl.BlockSpec((BN, ENC), lambda i: (i, 0))
    wspec = pl.BlockSpec((ENC, ENC), lambda i: (0, 0))
    bspec = pl.BlockSpec((1, ENC), lambda i: (0, 0))
    return pl.pallas_call(
        _node_mlp_tab_body,
        grid=grid,
        in_specs=[nspec] * 6 + [wspec] * 4 + [bspec, wspec, bspec]
                 + [wspec] * 4,
        out_specs=[nspec, nspec, nspec],
        out_shape=[jax.ShapeDtypeStruct((N_NODES, ENC), F32)] * 3,
    )(xc, x0, sa0, sa1, sb0, sb1, wA, wB, wC, wD, b1, w2, b2,
      wsc, ws0, wdc, wd0)


def _node_head_body(xc_ref, x0_ref, sa0_ref, sa1_ref, sb0_ref, sb1_ref,
                    wA_ref, wB_ref, wC_ref, wD_ref, b1_ref, w2_ref, b2_ref,
                    dW1_ref, db1_ref, dW2t_ref, db2_ref, out_ref):
    xcn = _node_core(xc_ref, x0_ref, sa0_ref, sa1_ref, sb0_ref, sb1_ref,
                     wA_ref, wB_ref, wC_ref, wD_ref, b1_ref, w2_ref, b2_ref)
    h2 = _relu(_dot(xcn, dW1_ref[...]) + db1_ref[...])
    out_ref[...] = (jnp.sum(h2 * dW2t_ref[...], axis=1, keepdims=True)
                    + db2_ref[...])


def _node_head(xc, x0, sa0, sa1, sb0, sb1, wA, wB, wC, wD, b1, w2, b2,
               dW1, db1, dW2t, db2):
    grid = (N_NODES // BN,)
    nspec = pl.BlockSpec((BN, ENC), lambda i: (i, 0))
    wspec = pl.BlockSpec((ENC, ENC), lambda i: (0, 0))
    bspec = pl.BlockSpec((1, ENC), lambda i: (0, 0))
    return pl.pallas_call(
        _node_head_body,
        grid=grid,
        in_specs=[nspec] * 6 + [wspec] * 4 + [bspec, wspec, bspec]
                 + [wspec, bspec, bspec,
                    pl.BlockSpec((1, 1), lambda i: (0, 0))],
        out_specs=pl.BlockSpec((BN, 1), lambda i: (i, 0)),
        out_shape=jax.ShapeDtypeStruct((N_NODES, 1), F32),
    )(xc, x0, sa0, sa1, sb0, sb1, wA, wB, wC, wD, b1, w2, b2,
      dW1, db1, dW2t, db2)


def _sc_gather_scatter(ps, pd, src, dst3, ec):
    mesh = plsc.VectorSubcoreMesh(core_axis_name="c", subcore_axis_name="s")

    @functools.partial(
        pl.kernel, mesh=mesh,
        out_type=(jax.ShapeDtypeStruct((N_EDGES, ENC), F32),
                  jax.ShapeDtypeStruct((N_EDGES, ENC), F32),
                  jax.ShapeDtypeStruct((NC * NP, ENC), F32)),
        scratch_types=[
            pltpu.VMEM((PER_W,), jnp.int32),
            pltpu.VMEM((NCHM, CHM), jnp.int32),
            pltpu.VMEM((UBM, CHM, ENC), F32),
            pltpu.VMEM((UBM, CHM, ENC), F32),
            pltpu.VMEM((UBM, CHM, ENC), F32),
            pltpu.VMEM((CHM, ENC), F32),
            pltpu.VMEM_SHARED((NP, ENC), F32),
            [pltpu.SemaphoreType.DMA] * UBM,
            [pltpu.SemaphoreType.DMA] * UBM,
            [pltpu.SemaphoreType.DMA] * UBM,
            pltpu.SemaphoreType.DMA,
            pltpu.SemaphoreType.DMA,
        ],
        compiler_params=pltpu.CompilerParams(use_tc_tiling_on_sc=False),
    )
    def k(ps_hbm, pd_hbm, src_hbm, dst3_hbm, ec_hbm,
          gs_hbm, gd_hbm, out_hbm,
          si_v, di_v, ra_v, rb_v, re_v, zb_v, acc_sh,
          sas, sbs, ses, sw, sv):
        c = lax.axis_index("c")
        s = lax.axis_index("s")
        w = c * NS + s
        base0 = w * PER_W
        zero16 = jnp.zeros((16,), F32)

        def zb_body(i, carry):
            for g in range(4):
                zb_v[i, pl.ds(g * 16, 16)] = zero16
            return carry

        lax.fori_loop(0, CHM, zb_body, 0)
        for t in range(16):
            pltpu.sync_copy(zb_v, acc_sh.at[pl.ds(s * 640 + t * CHM, CHM)])
        pltpu.sync_copy(src_hbm.at[pl.ds(base0, PER_W)], si_v)
        pltpu.sync_copy(dst3_hbm.at[w], di_v)
        plsc.subcore_barrier()

        def body(i, carry):
            j0 = i * UBM
            ga = []
            gb = []
            ld = []
            for u in range(UBM):
                j = j0 + u
                base = base0 + j * CHM
                ga.append(pltpu.async_copy(
                    ps_hbm.at[si_v.at[pl.ds(j * CHM, CHM)]], ra_v.at[u],
                    sas[u]))
                gb.append(pltpu.async_copy(
                    pd_hbm.at[di_v.at[j]], rb_v.at[u], sbs[u]))
                ld.append(pltpu.async_copy(ec_hbm.at[pl.ds(base, CHM)],
                                           re_v.at[u], ses[u]))
            fin = []
            for u in range(UBM):
                j = j0 + u
                base = base0 + j * CHM
                ld[u].wait()
                fin.append(pltpu.async_copy(re_v.at[u], acc_sh.at[di_v.at[j]],
                                            sv, add=True))
                ga[u].wait()
                fin.append(pltpu.async_copy(ra_v.at[u],
                                            gs_hbm.at[pl.ds(base, CHM)], sw))
                gb[u].wait()
                fin.append(pltpu.async_copy(rb_v.at[u],
                                            gd_hbm.at[pl.ds(base, CHM)], sw))
            for h in fin:
                h.wait()
            return carry

        lax.fori_loop(0, NCHM // UBM, body, 0)
        plsc.subcore_barrier()
        for t in range(16):
            r0 = s * 640 + t * CHM
            pltpu.sync_copy(acc_sh.at[pl.ds(r0, CHM)], zb_v)
            pltpu.sync_copy(zb_v, out_hbm.at[pl.ds(c * NP + r0, CHM)])

    return k(ps, pd, src, dst3, ec)


def _sc_gather(ps, pd, src, dst):
    mesh = plsc.VectorSubcoreMesh(core_axis_name="c", subcore_axis_name="s")

    @functools.partial(
        pl.kernel, mesh=mesh,
        out_type=(jax.ShapeDtypeStruct((N_EDGES, ENC), F32),
                  jax.ShapeDtypeStruct((N_EDGES, ENC), F32)),
        scratch_types=[
            pltpu.VMEM((PER_W,), jnp.int32),
            pltpu.VMEM((PER_W,), jnp.int32),
            pltpu.VMEM((UB, CH, ENC), F32),
            pltpu.VMEM((UB, CH, ENC), F32),
            [pltpu.SemaphoreType.DMA] * UB,
            [pltpu.SemaphoreType.DMA] * UB,
            pltpu.SemaphoreType.DMA,
        ],
        compiler_params=pltpu.CompilerParams(use_tc_tiling_on_sc=False),
    )
    def k(ps_hbm, pd_hbm, src_hbm, dst_hbm, gs_hbm, gd_hbm,
          si_v, di_v, ra_v, rb_v, sas, sbs, sw):
        c = lax.axis_index("c")
        s = lax.axis_index("s")
        w = s * NC + c
        base0 = w * PER_W
        pltpu.sync_copy(src_hbm.at[pl.ds(base0, PER_W)], si_v)
        pltpu.sync_copy(dst_hbm.at[pl.ds(base0, PER_W)], di_v)

        def body(i, carry):
            j0 = i * UB
            ga = []
            gb = []
            for u in range(UB):
                j = j0 + u
                ga.append(pltpu.async_copy(
                    ps_hbm.at[si_v.at[pl.ds(j * CH, CH)]], ra_v.at[u],
                    sas[u]))
                gb.append(pltpu.async_copy(
                    pd_hbm.at[di_v.at[pl.ds(j * CH, CH)]], rb_v.at[u],
                    sbs[u]))
            wr = []
            for u in range(UB):
                base = base0 + (j0 + u) * CH
                ga[u].wait()
                wr.append(pltpu.async_copy(
                    ra_v.at[u], gs_hbm.at[pl.ds(base, CH)], sw))
                gb[u].wait()
                wr.append(pltpu.async_copy(
                    rb_v.at[u], gd_hbm.at[pl.ds(base, CH)], sw))
            for h in wr:
                h.wait()
            return carry

        lax.fori_loop(0, NCH // UB, body, 0)

    return k(ps, pd, src, dst)


def _sc_scatter(ec, dst3):
    mesh = plsc.VectorSubcoreMesh(core_axis_name="c", subcore_axis_name="s")

    @functools.partial(
        pl.kernel, mesh=mesh,
        out_type=jax.ShapeDtypeStruct((NC * NP, ENC), F32),
        scratch_types=[
            pltpu.VMEM((NCH, CH), jnp.int32),
            pltpu.VMEM((UB, CH, ENC), F32),
            pltpu.VMEM((ZR, ENC), F32),
            pltpu.VMEM_SHARED((NP, ENC), F32),
            [pltpu.SemaphoreType.DMA] * UB,
            pltpu.SemaphoreType.DMA,
        ],
        compiler_params=pltpu.CompilerParams(use_tc_tiling_on_sc=False),
    )
    def k(ec_hbm, dst3_hbm, out_hbm, di_v, rows_v, zb_v, acc_sh, srs, sw):
        c = lax.axis_index("c")
        s = lax.axis_index("s")
        w = c * NS + s
        zero16 = jnp.zeros((16,), F32)

        def zb_body(i, carry):
            for g in range(4):
                zb_v[i, pl.ds(g * 16, 16)] = zero16
            return carry

        lax.fori_loop(0, ZR, zb_body, 0)
        for t in range(5):
            pltpu.sync_copy(zb_v, acc_sh.at[pl.ds((s * 5 + t) * ZR, ZR)])
        pltpu.sync_copy(dst3_hbm.at[w], di_v)
        plsc.subcore_barrier()

        base0 = w * PER_W

        def body(i, carry):
            j0 = i * UB
            ld = []
            for u in range(UB):
                base = base0 + (j0 + u) * CH
                ld.append(pltpu.async_copy(
                    ec_hbm.at[pl.ds(base, CH)], rows_v.at[u], srs[u]))
            upd = []
            for u in range(UB):
                ld[u].wait()
                upd.append(pltpu.async_copy(
                    rows_v.at[u], acc_sh.at[di_v.at[j0 + u]], sw, add=True))
            for h in upd:
                h.wait()
            return carry

        lax.fori_loop(0, NCH // UB, body, 0)
        plsc.subcore_barrier()
        for t in range(5):
            r0 = (s * 5 + t) * ZR
            pltpu.sync_copy(acc_sh.at[pl.ds(r0, ZR)], zb_v)
            pltpu.sync_copy(zb_v, out_hbm.at[pl.ds(c * NP + r0, ZR)])

    return k(ec, dst3)


def kernel(x, edge_features, edge_index, nW1, nb1, nW2, nb2, eW1, eb1, eW2,
           eb2, mWe1, mbe1, mWe2, mbe2, mWn1, mbn1, mWn2, mbn2, dW1, db1,
           dW2, db2):
    src = edge_index[0]
    dst = edge_index[1]
    dst3m = dst.reshape(NW, NCHM, CHM)
    dst3 = dst.reshape(NW, NCH, CH)

    E = ENC
    nb1r = nb1.reshape(1, E)
    nb2r = nb2.reshape(1, E)
    eb1r = eb1.reshape(1, E)
    eb2r = eb2.reshape(1, E)
    db1r = db1.reshape(1, E)
    db2r = db2.reshape(1, 1)
    dW2t = dW2.reshape(1, E)

    def we(l, k):
        return mWe1[l, k * E:(k + 1) * E, :]

    def wn(l, k):
        return mWn1[l, k * E:(k + 1) * E, :]

    x0, ps0, pd0 = _node_encoder(x, nW1, nb1r, nW2, nb2r,
                                 we(0, 2) + we(0, 3), we(0, 4) + we(0, 5))
    e0 = _edge_encoder(edge_features, eW1, eb1r, eW2, eb2r)

    def _parts(s):
        return s[:N_NODES], s[NP:NP + N_NODES]

    gs0, gd0, sp0 = _sc_gather_scatter(ps0, pd0, src, dst3m, e0)
    s0a, s0b = _parts(sp0)
    ec1 = _edge_mlp0(e0, gs0, gd0, we(0, 0) + we(0, 1),
                     mbe1[0].reshape(1, E), mWe2[0], mbe2[0].reshape(1, E))
    xc1, ps1, pd1 = _node_mlp_tab(
        x0, x0, s0a, s0b, s0a, s0b,
        wn(0, 0), wn(0, 1), wn(0, 2), wn(0, 3),
        mbn1[0].reshape(1, E), mWn2[0], mbn2[0].reshape(1, E),
        we(1, 2), we(1, 3), we(1, 4), we(1, 5))

    gs1, gd1, sp1 = _sc_gather_scatter(ps1, pd1, src, dst3m, ec1)
    s1a, s1b = _parts(sp1)
    ec2 = _edge_mlp(ec1, e0, gs1, gd1, we(1, 0), we(1, 1),
                    mbe1[1].reshape(1, E), mWe2[1], mbe2[1].reshape(1, E))
    xc2, _, _ = _node_mlp_tab(
        xc1, x0, s1a, s1b, s0a, s0b,
        wn(1, 0), wn(1, 1), wn(1, 2), wn(1, 3),
        mbn1[1].reshape(1, E), mWn2[1], mbn2[1].reshape(1, E),
        we(2, 2), we(2, 3), we(2, 4), we(2, 5))

    s2a, s2b = _parts(_sc_scatter(ec2, dst3))
    out = _node_head(
        xc2, x0, s2a, s2b, s0a, s0b,
        wn(2, 0), wn(2, 1), wn(2, 2), wn(2, 3),
        mbn1[2].reshape(1, E), mWn2[2], mbn2[2].reshape(1, E),
        dW1, db1r, dW2t, db2r)
    return out


# speedup vs baseline: 1.0884x; 1.0884x over previous
import functools

import jax
import jax.numpy as jnp
from jax import lax
from jax.experimental import pallas as pl
from jax.experimental.pallas import tpu as pltpu
from jax.experimental.pallas import tpu_sc as plsc

N_NODES = 10000
N_EDGES = 320000
ENC = 64
F32 = jnp.float32

BE = 4000
BN = 2000

NC = 2
NS = 16
NW = NC * NS
CH = 80
PER_W = N_EDGES // NW
NCH = PER_W // CH
UB = 5
CHM = 40
NCHM = PER_W // CHM
UBM = 2
NP = 10240
ZR = 128


def _relu(v):
    return jnp.maximum(v, 0.0)


BF16 = jnp.bfloat16


def _dot(a, b):
    return jnp.dot(a.astype(BF16), b.astype(BF16), preferred_element_type=F32)


def _rnd(a):
    return a.astype(BF16).astype(F32)


def _node_enc_body(x_ref, nW1_ref, nb1_ref, nW2_ref, nb2_ref,
                   wsc_ref, ws0_ref, wdc_ref, wd0_ref,
                   x0_ref, ps_ref, pd_ref):
    xb = _rnd(x_ref[...])
    w1 = _rnd(nW1_ref[...])
    h = (xb[:, 0:1] * w1[0:1, :] + xb[:, 1:2] * w1[1:2, :]
         + nb1_ref[...])
    x0 = _relu(_dot(_relu(h), nW2_ref[...]) + nb2_ref[...])
    x0_ref[...] = x0
    ps_ref[...] = _dot(x0, wsc_ref[...]) + _dot(x0, ws0_ref[...])
    pd_ref[...] = _dot(x0, wdc_ref[...]) + _dot(x0, wd0_ref[...])


def _node_encoder(x, nW1, nb1, nW2, nb2, wsc, ws0, wdc, wd0):
    grid = (N_NODES // BN,)
    wspec = pl.BlockSpec((ENC, ENC), lambda i: (0, 0))
    bspec = pl.BlockSpec((1, ENC), lambda i: (0, 0))
    nspec = pl.BlockSpec((BN, ENC), lambda i: (i, 0))
    return pl.pallas_call(
        _node_enc_body,
        grid=grid,
        in_specs=[pl.BlockSpec((BN, 2), lambda i: (i, 0)),
                  pl.BlockSpec((2, ENC), lambda i: (0, 0)), bspec,
                  wspec, bspec, wspec, wspec, wspec, wspec],
        out_specs=[nspec, nspec, nspec],
        out_shape=[jax.ShapeDtypeStruct((N_NODES, ENC), F32)] * 3,
    )(x, nW1, nb1, nW2, nb2, wsc, ws0, wdc, wd0)


def _edge_enc_body(ef_ref, eW1_ref, eb1_ref, eW2_ref, eb2_ref, e0_ref):
    eb = _rnd(ef_ref[...])
    w1 = _rnd(eW1_ref[...])
    h = (eb[:, 0:1] * w1[0:1, :] + eb[:, 1:2] * w1[1:2, :]
         + eb[:, 2:3] * w1[2:3, :] + eb1_ref[...])
    e0_ref[...] = _relu(_dot(_relu(h), eW2_ref[...]) + eb2_ref[...])


def _edge_encoder(ef, eW1, eb1, eW2, eb2):
    grid = (N_EDGES // BE,)
    return pl.pallas_call(
        _edge_enc_body,
        grid=grid,
        in_specs=[pl.BlockSpec((BE, 3), lambda i: (i, 0)),
                  pl.BlockSpec((3, ENC), lambda i: (0, 0)),
                  pl.BlockSpec((1, ENC), lambda i: (0, 0)),
                  pl.BlockSpec((ENC, ENC), lambda i: (0, 0)),
                  pl.BlockSpec((1, ENC), lambda i: (0, 0))],
        out_specs=pl.BlockSpec((BE, ENC), lambda i: (i, 0)),
        out_shape=jax.ShapeDtypeStruct((N_EDGES, ENC), F32),
    )(ef, eW1, eb1, eW2, eb2)


def _edge_mlp0_body(e0_ref, gs_ref, gd_ref, w1c_ref, w1e_ref, b1_ref,
                    w2_ref, b2_ref, out_ref):
    e0 = e0_ref[...]
    h = _relu(_dot(e0, w1c_ref[...]) + _dot(e0, w1e_ref[...]) + gs_ref[...]
              + gd_ref[...] + b1_ref[...])
    out_ref[...] = _relu(_dot(h, w2_ref[...]) + b2_ref[...])


def _edge_mlp0(e0, gs, gd, w1c, w1e, b1, w2, b2):
    grid = (N_EDGES // BE,)
    espec = pl.BlockSpec((BE, ENC), lambda i: (i, 0))
    wspec = pl.BlockSpec((ENC, ENC), lambda i: (0, 0))
    bspec = pl.BlockSpec((1, ENC), lambda i: (0, 0))
    return pl.pallas_call(
        _edge_mlp0_body,
        grid=grid,
        in_specs=[espec, espec, espec, wspec, wspec, bspec, wspec, bspec],
        out_specs=espec,
        out_shape=jax.ShapeDtypeStruct((N_EDGES, ENC), F32),
    )(e0, gs, gd, w1c, w1e, b1, w2, b2)


def _edge_mlp_body(ec_ref, e0_ref, gs_ref, gd_ref, w1c_ref, w1e_ref, b1_ref,
                   w2_ref, b2_ref, out_ref):
    h = _relu(_dot(ec_ref[...], w1c_ref[...]) + _dot(e0_ref[...], w1e_ref[...])
              + gs_ref[...] + gd_ref[...] + b1_ref[...])
    out_ref[...] = _relu(_dot(h, w2_ref[...]) + b2_ref[...])


def _edge_mlp(ec, e0, gs, gd, w1c, w1e, b1, w2, b2):
    grid = (N_EDGES // BE,)
    espec = pl.BlockSpec((BE, ENC), lambda i: (i, 0))
    wspec = pl.BlockSpec((ENC, ENC), lambda i: (0, 0))
    bspec = pl.BlockSpec((1, ENC), lambda i: (0, 0))
    return pl.pallas_call(
        _edge_mlp_body,
        grid=grid,
        in_specs=[espec, espec, espec, espec, wspec, wspec, bspec, wspec,
                  bspec],
        out_specs=espec,
        out_shape=jax.ShapeDtypeStruct((N_EDGES, ENC), F32),
    )(ec, e0, gs, gd, w1c, w1e, b1, w2, b2)


def _node_core(xc_ref, x0_ref, sa0_ref, sa1_ref, sb0_ref, sb1_ref,
               wA_ref, wB_ref, wC_ref, wD_ref, b1_ref, w2_ref, b2_ref):
    sec = sa0_ref[...] + sa1_ref[...]
    se0 = sb0_ref[...] + sb1_ref[...]
    h = _relu(_dot(xc_ref[...], wA_ref[...]) + _dot(x0_ref[...], wB_ref[...])
              + _dot(sec, wC_ref[...]) + _dot(se0, wD_ref[...]) + b1_ref[...])
    return _relu(_dot(h, w2_ref[...]) + b2_ref[...])


def _node_mlp_tab_body(xc_ref, x0_ref, sa0_ref, sa1_ref, sb0_ref, sb1_ref,
                       wA_ref, wB_ref, wC_ref, wD_ref, b1_ref, w2_ref, b2_ref,
                       wsc_ref, ws0_ref, wdc_ref, wd0_ref,
                       xcn_ref, ps_ref, pd_ref):
    xcn = _node_core(xc_ref, x0_ref, sa0_ref, sa1_ref, sb0_ref, sb1_ref,
                     wA_ref, wB_ref, wC_ref, wD_ref, b1_ref, w2_ref, b2_ref)
    xcn_ref[...] = xcn
    x0 = x0_ref[...]
    ps_ref[...] = _dot(xcn, wsc_ref[...]) + _dot(x0, ws0_ref[...])
    pd_ref[...] = _dot(xcn, wdc_ref[...]) + _dot(x0, wd0_ref[...])


def _node_mlp_tab(xc, x0, sa0, sa1, sb0, sb1, wA, wB, wC, wD, b1, w2, b2,
                  wsc, ws0, wdc, wd0):
    grid = (N_NODES // BN,)
    nspec = pl.BlockSpec((BN, ENC), lambda i: (i, 0))
    wspec = pl.BlockSpec((ENC, ENC), lambda i: (0, 0))
    bspec = pl.BlockSpec((1, ENC), lambda i: (0, 0))
    return pl.pallas_call(
        _node_mlp_tab_body,
        grid=grid,
        in_specs=[nspec] * 6 + [wspec] * 4 + [bspec, wspec, bspec]
                 + [wspec] * 4,
        out_specs=[nspec, nspec, nspec],
        out_shape=[jax.ShapeDtypeStruct((N_NODES, ENC), F32)] * 3,
    )(xc, x0, sa0, sa1, sb0, sb1, wA, wB, wC, wD, b1, w2, b2,
      wsc, ws0, wdc, wd0)


def _node_head_body(xc_ref, x0_ref, sa0_ref, sa1_ref, sb0_ref, sb1_ref,
                    wA_ref, wB_ref, wC_ref, wD_ref, b1_ref, w2_ref, b2_ref,
                    dW1_ref, db1_ref, dW2t_ref, db2_ref, out_ref):
    xcn = _node_core(xc_ref, x0_ref, sa0_ref, sa1_ref, sb0_ref, sb1_ref,
                     wA_ref, wB_ref, wC_ref, wD_ref, b1_ref, w2_ref, b2_ref)
    h2 = _relu(_dot(xcn, dW1_ref[...]) + db1_ref[...])
    out_ref[...] = (jnp.sum(_rnd(h2) * _rnd(dW2t_ref[...]), axis=1,
                            keepdims=True) + db2_ref[...])


def _node_head(xc, x0, sa0, sa1, sb0, sb1, wA, wB, wC, wD, b1, w2, b2,
               dW1, db1, dW2t, db2):
    grid = (N_NODES // BN,)
    nspec = pl.BlockSpec((BN, ENC), lambda i: (i, 0))
    wspec = pl.BlockSpec((ENC, ENC), lambda i: (0, 0))
    bspec = pl.BlockSpec((1, ENC), lambda i: (0, 0))
    return pl.pallas_call(
        _node_head_body,
        grid=grid,
        in_specs=[nspec] * 6 + [wspec] * 4 + [bspec, wspec, bspec]
                 + [wspec, bspec, bspec,
                    pl.BlockSpec((1, 1), lambda i: (0, 0))],
        out_specs=pl.BlockSpec((BN, 1), lambda i: (i, 0)),
        out_shape=jax.ShapeDtypeStruct((N_NODES, 1), F32),
    )(xc, x0, sa0, sa1, sb0, sb1, wA, wB, wC, wD, b1, w2, b2,
      dW1, db1, dW2t, db2)


def _sc_gather_scatter(ps, pd, src, dst3, ec):
    mesh = plsc.VectorSubcoreMesh(core_axis_name="c", subcore_axis_name="s")

    @functools.partial(
        pl.kernel, mesh=mesh,
        out_type=(jax.ShapeDtypeStruct((N_EDGES, ENC), F32),
                  jax.ShapeDtypeStruct((N_EDGES, ENC), F32),
                  jax.ShapeDtypeStruct((NC * NP, ENC), F32)),
        scratch_types=[
            pltpu.VMEM((PER_W,), jnp.int32),
            pltpu.VMEM((NCHM, CHM), jnp.int32),
            pltpu.VMEM((UBM, CHM, ENC), F32),
            pltpu.VMEM((UBM, CHM, ENC), F32),
            pltpu.VMEM((UBM, CHM, ENC), F32),
            pltpu.VMEM((CHM, ENC), F32),
            pltpu.VMEM_SHARED((NP, ENC), F32),
            [pltpu.SemaphoreType.DMA] * UBM,
            [pltpu.SemaphoreType.DMA] * UBM,
            [pltpu.SemaphoreType.DMA] * UBM,
            pltpu.SemaphoreType.DMA,
            pltpu.SemaphoreType.DMA,
        ],
        compiler_params=pltpu.CompilerParams(use_tc_tiling_on_sc=False),
    )
    def k(ps_hbm, pd_hbm, src_hbm, dst3_hbm, ec_hbm,
          gs_hbm, gd_hbm, out_hbm,
          si_v, di_v, ra_v, rb_v, re_v, zb_v, acc_sh,
          sas, sbs, ses, sw, sv):
        c = lax.axis_index("c")
        s = lax.axis_index("s")
        w = c * NS + s
        base0 = w * PER_W
        zero16 = jnp.zeros((16,), F32)

        def zb_body(i, carry):
            for g in range(4):
                zb_v[i, pl.ds(g * 16, 16)] = zero16
            return carry

        lax.fori_loop(0, CHM, zb_body, 0)
        for t in range(16):
            pltpu.sync_copy(zb_v, acc_sh.at[pl.ds(s * 640 + t * CHM, CHM)])
        pltpu.sync_copy(src_hbm.at[pl.ds(base0, PER_W)], si_v)
        pltpu.sync_copy(dst3_hbm.at[w], di_v)
        plsc.subcore_barrier()

        def body(i, carry):
            j0 = i * UBM
            ga = []
            gb = []
            ld = []
            for u in range(UBM):
                j = j0 + u
                base = base0 + j * CHM
                ga.append(pltpu.async_copy(
                    ps_hbm.at[si_v.at[pl.ds(j * CHM, CHM)]], ra_v.at[u],
                    sas[u]))
                gb.append(pltpu.async_copy(
                    pd_hbm.at[di_v.at[j]], rb_v.at[u], sbs[u]))
                ld.append(pltpu.async_copy(ec_hbm.at[pl.ds(base, CHM)],
                                           re_v.at[u], ses[u]))
            fin = []
            for u in range(UBM):
                j = j0 + u
                base = base0 + j * CHM
                ld[u].wait()
                fin.append(pltpu.async_copy(re_v.at[u], acc_sh.at[di_v.at[j]],
                                            sv, add=True))
                ga[u].wait()
                fin.append(pltpu.async_copy(ra_v.at[u],
                                            gs_hbm.at[pl.ds(base, CHM)], sw))
                gb[u].wait()
                fin.append(pltpu.async_copy(rb_v.at[u],
                                            gd_hbm.at[pl.ds(base, CHM)], sw))
            for h in fin:
                h.wait()
            return carry

        lax.fori_loop(0, NCHM // UBM, body, 0)
        plsc.subcore_barrier()
        for t in range(16):
            r0 = s * 640 + t * CHM
            pltpu.sync_copy(acc_sh.at[pl.ds(r0, CHM)], zb_v)
            pltpu.sync_copy(zb_v, out_hbm.at[pl.ds(c * NP + r0, CHM)])

    return k(ps, pd, src, dst3, ec)


def _sc_gather(ps, pd, src, dst):
    mesh = plsc.VectorSubcoreMesh(core_axis_name="c", subcore_axis_name="s")

    @functools.partial(
        pl.kernel, mesh=mesh,
        out_type=(jax.ShapeDtypeStruct((N_EDGES, ENC), F32),
                  jax.ShapeDtypeStruct((N_EDGES, ENC), F32)),
        scratch_types=[
            pltpu.VMEM((PER_W,), jnp.int32),
            pltpu.VMEM((PER_W,), jnp.int32),
            pltpu.VMEM((UB, CH, ENC), F32),
            pltpu.VMEM((UB, CH, ENC), F32),
            [pltpu.SemaphoreType.DMA] * UB,
            [pltpu.SemaphoreType.DMA] * UB,
            pltpu.SemaphoreType.DMA,
        ],
        compiler_params=pltpu.CompilerParams(use_tc_tiling_on_sc=False),
    )
    def k(ps_hbm, pd_hbm, src_hbm, dst_hbm, gs_hbm, gd_hbm,
          si_v, di_v, ra_v, rb_v, sas, sbs, sw):
        c = lax.axis_index("c")
        s = lax.axis_index("s")
        w = s * NC + c
        base0 = w * PER_W
        pltpu.sync_copy(src_hbm.at[pl.ds(base0, PER_W)], si_v)
        pltpu.sync_copy(dst_hbm.at[pl.ds(base0, PER_W)], di_v)

        def body(i, carry):
            j0 = i * UB
            ga = []
            gb = []
            for u in range(UB):
                j = j0 + u
                ga.append(pltpu.async_copy(
                    ps_hbm.at[si_v.at[pl.ds(j * CH, CH)]], ra_v.at[u],
                    sas[u]))
                gb.append(pltpu.async_copy(
                    pd_hbm.at[di_v.at[pl.ds(j * CH, CH)]], rb_v.at[u],
                    sbs[u]))
            wr = []
            for u in range(UB):
                base = base0 + (j0 + u) * CH
                ga[u].wait()
                wr.append(pltpu.async_copy(
                    ra_v.at[u], gs_hbm.at[pl.ds(base, CH)], sw))
                gb[u].wait()
                wr.append(pltpu.async_copy(
                    rb_v.at[u], gd_hbm.at[pl.ds(base, CH)], sw))
            for h in wr:
                h.wait()
            return carry

        lax.fori_loop(0, NCH // UB, body, 0)

    return k(ps, pd, src, dst)


def _sc_scatter(ec, dst3):
    mesh = plsc.VectorSubcoreMesh(core_axis_name="c", subcore_axis_name="s")

    @functools.partial(
        pl.kernel, mesh=mesh,
        out_type=jax.ShapeDtypeStruct((NC * NP, ENC), F32),
        scratch_types=[
            pltpu.VMEM((NCH, CH), jnp.int32),
            pltpu.VMEM((UB, CH, ENC), F32),
            pltpu.VMEM((ZR, ENC), F32),
            pltpu.VMEM_SHARED((NP, ENC), F32),
            [pltpu.SemaphoreType.DMA] * UB,
            pltpu.SemaphoreType.DMA,
        ],
        compiler_params=pltpu.CompilerParams(use_tc_tiling_on_sc=False),
    )
    def k(ec_hbm, dst3_hbm, out_hbm, di_v, rows_v, zb_v, acc_sh, srs, sw):
        c = lax.axis_index("c")
        s = lax.axis_index("s")
        w = c * NS + s
        zero16 = jnp.zeros((16,), F32)

        def zb_body(i, carry):
            for g in range(4):
                zb_v[i, pl.ds(g * 16, 16)] = zero16
            return carry

        lax.fori_loop(0, ZR, zb_body, 0)
        for t in range(5):
            pltpu.sync_copy(zb_v, acc_sh.at[pl.ds((s * 5 + t) * ZR, ZR)])
        pltpu.sync_copy(dst3_hbm.at[w], di_v)
        plsc.subcore_barrier()

        base0 = w * PER_W

        def body(i, carry):
            j0 = i * UB
            ld = []
            for u in range(UB):
                base = base0 + (j0 + u) * CH
                ld.append(pltpu.async_copy(
                    ec_hbm.at[pl.ds(base, CH)], rows_v.at[u], srs[u]))
            upd = []
            for u in range(UB):
                ld[u].wait()
                upd.append(pltpu.async_copy(
                    rows_v.at[u], acc_sh.at[di_v.at[j0 + u]], sw, add=True))
            for h in upd:
                h.wait()
            return carry

        lax.fori_loop(0, NCH // UB, body, 0)
        plsc.subcore_barrier()
        for t in range(5):
            r0 = (s * 5 + t) * ZR
            pltpu.sync_copy(acc_sh.at[pl.ds(r0, ZR)], zb_v)
            pltpu.sync_copy(zb_v, out_hbm.at[pl.ds(c * NP + r0, ZR)])

    return k(ec, dst3)


def kernel(x, edge_features, edge_index, nW1, nb1, nW2, nb2, eW1, eb1, eW2,
           eb2, mWe1, mbe1, mWe2, mbe2, mWn1, mbn1, mWn2, mbn2, dW1, db1,
           dW2, db2):
    src = edge_index[0]
    dst = edge_index[1]
    dst3 = dst.reshape(NW, NCH, CH)

    E = ENC
    nb1r = nb1.reshape(1, E)
    nb2r = nb2.reshape(1, E)
    eb1r = eb1.reshape(1, E)
    eb2r = eb2.reshape(1, E)
    db1r = db1.reshape(1, E)
    db2r = db2.reshape(1, 1)
    dW2t = dW2.reshape(1, E)

    def we(l, k):
        return mWe1[l, k * E:(k + 1) * E, :]

    def wn(l, k):
        return mWn1[l, k * E:(k + 1) * E, :]

    x0, ps0, pd0 = _node_encoder(x, nW1, nb1r, nW2, nb2r,
                                 we(0, 2), we(0, 3), we(0, 4), we(0, 5))
    e0 = _edge_encoder(edge_features, eW1, eb1r, eW2, eb2r)

    def _parts(s):
        return s[:N_NODES], s[NP:NP + N_NODES]

    s0a, s0b = _parts(_sc_scatter(e0, dst3))
    gs0, gd0 = _sc_gather(ps0, pd0, src, dst)
    ec1 = _edge_mlp0(e0, gs0, gd0, we(0, 0), we(0, 1),
                     mbe1[0].reshape(1, E), mWe2[0], mbe2[0].reshape(1, E))
    xc1, ps1, pd1 = _node_mlp_tab(
        x0, x0, s0a, s0b, s0a, s0b,
        wn(0, 0), wn(0, 1), wn(0, 2), wn(0, 3),
        mbn1[0].reshape(1, E), mWn2[0], mbn2[0].reshape(1, E),
        we(1, 2), we(1, 3), we(1, 4), we(1, 5))

    s1a, s1b = _parts(_sc_scatter(ec1, dst3))
    gs1, gd1 = _sc_gather(ps1, pd1, src, dst)
    ec2 = _edge_mlp(ec1, e0, gs1, gd1, we(1, 0), we(1, 1),
                    mbe1[1].reshape(1, E), mWe2[1], mbe2[1].reshape(1, E))
    xc2, _, _ = _node_mlp_tab(
        xc1, x0, s1a, s1b, s0a, s0b,
        wn(1, 0), wn(1, 1), wn(1, 2), wn(1, 3),
        mbn1[1].reshape(1, E), mWn2[1], mbn2[1].reshape(1, E),
        we(2, 2), we(2, 3), we(2, 4), we(2, 5))

    s2a, s2b = _parts(_sc_scatter(ec2, dst3))
    out = _node_head(
        xc2, x0, s2a, s2b, s0a, s0b,
        wn(2, 0), wn(2, 1), wn(2, 2), wn(2, 3),
        mbn1[2].reshape(1, E), mWn2[2], mbn2[2].reshape(1, E),
        dW1, db1r, dW2t, db2r)
    return out
